# baseline (speedup 1.0000x reference)
import functools

import jax
import jax.numpy as jnp
from jax import lax
from jax.experimental import pallas as pl
from jax.experimental.pallas import tpu as pltpu
from jax.experimental.pallas import tpu_sc as plsc

NUM_EMB = 100000
D = 64
B_ROWS = 16384
B_COLS = 200

NW = 32
BPW = B_ROWS // NW
NBB = BPW // 128
NTG = 8
NG = NBB * (B_COLS // NTG)

_mesh = plsc.VectorSubcoreMesh(core_axis_name="c", subcore_axis_name="s")


@functools.partial(
    pl.kernel,
    mesh=_mesh,
    compiler_params=pltpu.CompilerParams(use_tc_tiling_on_sc=False, needs_layout_passes=False),
    out_type=jax.ShapeDtypeStruct((B_COLS, 8, 128, 1024), jnp.float32),
    scratch_types=[
        pltpu.VMEM((NTG, 128), jnp.int32),
        pltpu.VMEM((128, D), jnp.float32),
        pltpu.VMEM((128, D), jnp.float32),
        pltpu.VMEM((8, 1024), jnp.float32),
        pltpu.VMEM((8, 1024), jnp.float32),
        pltpu.SemaphoreType.DMA,
        pltpu.SemaphoreType.DMA,
        pltpu.SemaphoreType.DMA,
        pltpu.SemaphoreType.DMA,
    ],
)
def _emb_lookup(xt_hbm, table_hbm, out_hbm, xblk, g0, g1, s0, s1,
                sem_g0, sem_g1, sem_s0, sem_s1):
    wid = lax.axis_index("s") * 2 + lax.axis_index("c")
    w_b0 = wid * BPW
    w_bt0 = wid * NBB
    lane = lax.iota(jnp.int32, 16)
    rows = [lane + (grp * 16) for grp in range(8)]

    def gather(tt, gb, sem):
        return pltpu.make_async_copy(
            table_hbm.at[xblk.at[tt]], gb, sem)

    def store(t, bt, sb, sem):
        return pltpu.make_async_copy(
            sb, out_hbm.at[t, pl.ds(0, 8), bt], sem)

    def transpose(gb, sb):
        def dbody(d4, carry):
            for u in range(4):
                d = d4 * 4 + u
                col = jnp.full((16,), d, jnp.int32)
                i0 = d // 8
                off = (d - (d // 8) * 8) * 128
                for grp in range(8):
                    v = plsc.load_gather(gb, [rows[grp], col])
                    sb[i0, pl.ds(off + grp * 16, 16)] = v
            return carry

        lax.fori_loop(0, D // 4, dbody, 0)

    def group(gidx, first):
        bb = gidx // (B_COLS // NTG)
        tg = gidx - bb * (B_COLS // NTG)
        t0 = tg * NTG
        b0 = w_b0 + bb * 128
        bt = w_bt0 + bb
        pltpu.sync_copy(
            xt_hbm.at[pl.ds(t0, NTG), pl.ds(b0, 128)], xblk)
        gather(0, g0, sem_g0).start()
        for tt in range(NTG):
            gb, sg = (g0, sem_g0) if tt % 2 == 0 else (g1, sem_g1)
            sb, ss = (s0, sem_s0) if tt % 2 == 0 else (s1, sem_s1)
            nb, sn = (g1, sem_g1) if tt % 2 == 0 else (g0, sem_g0)
            if tt + 1 < NTG:
                gather(tt + 1, nb, sn).start()
            gather(tt, gb, sg).wait()
            if not (first and tt < 2):
                store(0, 0, sb, ss).wait()
            transpose(gb, sb)
            store(t0 + tt, bt, sb, ss).start()

    group(0, True)

    def body(gidx, carry):
        group(gidx, False)
        return carry

    lax.fori_loop(1, NG, body, 0)

    store(0, 0, s0, sem_s0).wait()
    store(0, 0, s1, sem_s1).wait()


def kernel(x, weight):
    xt = jnp.transpose(x).astype(jnp.int32)
    a = _emb_lookup(xt, weight)
    a5 = a.reshape(B_COLS, 8, 128, 8, 128)
    return jnp.transpose(a5, (2, 4, 0, 1, 3)).reshape(B_ROWS, B_COLS, D)


# speedup vs baseline: 4.1392x; 4.1392x over previous
import functools

import jax
import jax.numpy as jnp
from jax import lax
from jax.experimental import pallas as pl
from jax.experimental.pallas import tpu as pltpu
from jax.experimental.pallas import tpu_sc as plsc

NUM_EMB = 100000
D = 64
B_ROWS = 16384
B_COLS = 200

NW = 32
BPW = B_ROWS // NW
NBB = BPW // 128
NTG = 8
NG = NBB * (B_COLS // NTG)

_mesh = plsc.VectorSubcoreMesh(core_axis_name="c", subcore_axis_name="s")


@functools.partial(
    pl.kernel,
    mesh=_mesh,
    compiler_params=pltpu.CompilerParams(use_tc_tiling_on_sc=False, needs_layout_passes=False),
    out_type=jax.ShapeDtypeStruct((B_COLS, 8, 128, 1024), jnp.float32),
    scratch_types=[
        pltpu.VMEM((NTG, 128), jnp.int32),
        pltpu.VMEM((128, D), jnp.float32),
        pltpu.VMEM((128, D), jnp.float32),
        pltpu.VMEM((8, 1024), jnp.float32),
        pltpu.VMEM((8, 1024), jnp.float32),
        pltpu.SemaphoreType.DMA,
        pltpu.SemaphoreType.DMA,
        pltpu.SemaphoreType.DMA,
        pltpu.SemaphoreType.DMA,
    ],
)
def _emb_lookup(xt_hbm, table_hbm, out_hbm, xblk, g0, g1, s0, s1,
                sem_g0, sem_g1, sem_s0, sem_s1):
    wid = lax.axis_index("s") * 2 + lax.axis_index("c")
    w_b0 = wid * BPW
    w_bt0 = wid * NBB
    lane = lax.iota(jnp.int32, 16)
    rows = [lane + (grp * 16) for grp in range(8)]

    def gather(tt, gb, sem):
        return pltpu.make_async_copy(
            table_hbm.at[xblk.at[tt]], gb, sem)

    def store(t, bt, sb, sem):
        return pltpu.make_async_copy(
            sb, out_hbm.at[t, pl.ds(0, 8), bt], sem)

    def transpose(gb, sb):
        @plsc.parallel_loop(0, D, unroll=4)
        def dbody(d):
            col = (jnp.full((16,), d, jnp.int32) + lane) & 63
            i0 = col >> 3
            i1base = (col & 7) << 7
            for grp in range(8):
                v = plsc.load_gather(gb, [rows[grp], col])
                plsc.store_scatter(sb, [i0, i1base + rows[grp]], v)

    def group(gidx, first):
        bb = gidx // (B_COLS // NTG)
        tg = gidx - bb * (B_COLS // NTG)
        t0 = tg * NTG
        b0 = w_b0 + bb * 128
        bt = w_bt0 + bb
        pltpu.sync_copy(
            xt_hbm.at[pl.ds(t0, NTG), pl.ds(b0, 128)], xblk)
        gather(0, g0, sem_g0).start()
        for tt in range(NTG):
            gb, sg = (g0, sem_g0) if tt % 2 == 0 else (g1, sem_g1)
            sb, ss = (s0, sem_s0) if tt % 2 == 0 else (s1, sem_s1)
            nb, sn = (g1, sem_g1) if tt % 2 == 0 else (g0, sem_g0)
            if tt + 1 < NTG:
                gather(tt + 1, nb, sn).start()
            gather(tt, gb, sg).wait()
            if not (first and tt < 2):
                store(0, 0, sb, ss).wait()
            transpose(gb, sb)
            store(t0 + tt, bt, sb, ss).start()

    group(0, True)

    def body(gidx, carry):
        group(gidx, False)
        return carry

    lax.fori_loop(1, NG, body, 0)

    store(0, 0, s0, sem_s0).wait()
    store(0, 0, s1, sem_s1).wait()


def kernel(x, weight):
    xt = jnp.transpose(x).astype(jnp.int32)
    a = _emb_lookup(xt, weight)
    a5 = a.reshape(B_COLS, 8, 128, 8, 128)
    return jnp.transpose(a5, (2, 4, 0, 1, 3)).reshape(B_ROWS, B_COLS, D)


# speedup vs baseline: 5.2313x; 1.2638x over previous
import functools

import jax
import jax.numpy as jnp
from jax import lax
from jax.experimental import pallas as pl
from jax.experimental.pallas import tpu as pltpu
from jax.experimental.pallas import tpu_sc as plsc

NUM_EMB = 100000
D = 64
B_ROWS = 16384
B_COLS = 200

NW = 32
BPW = B_ROWS // NW
NBB = BPW // 128
NTG = 8
NG = NBB * (B_COLS // NTG)

_mesh = plsc.VectorSubcoreMesh(core_axis_name="c", subcore_axis_name="s")


@functools.partial(
    pl.kernel,
    mesh=_mesh,
    compiler_params=pltpu.CompilerParams(use_tc_tiling_on_sc=False, needs_layout_passes=False),
    out_type=jax.ShapeDtypeStruct((B_COLS, 8, 128, 1024), jnp.float32),
    scratch_types=[
        pltpu.VMEM((NTG, 128), jnp.int32),
        pltpu.VMEM((128, D), jnp.float32),
        pltpu.VMEM((128, D), jnp.float32),
        pltpu.VMEM((8, 1024), jnp.float32),
        pltpu.VMEM((8, 1024), jnp.float32),
        pltpu.SemaphoreType.DMA,
        pltpu.SemaphoreType.DMA,
        pltpu.SemaphoreType.DMA,
        pltpu.SemaphoreType.DMA,
    ],
)
def _emb_lookup(xt_hbm, table_hbm, out_hbm, xblk, g0, g1, s0, s1,
                sem_g0, sem_g1, sem_s0, sem_s1):
    wid = lax.axis_index("s") * 2 + lax.axis_index("c")
    w_b0 = wid * BPW
    w_bt0 = wid * NBB
    lane = lax.iota(jnp.int32, 16)
    rows = [lane + (grp * 16) for grp in range(8)]

    def gather(tt, gb, sem):
        return pltpu.make_async_copy(
            table_hbm.at[xblk.at[tt]], gb, sem)

    def store(t, bt, sb, sem):
        return pltpu.make_async_copy(
            sb, out_hbm.at[t, pl.ds(0, 8), bt], sem)

    def transpose(gb, sb):
        @plsc.parallel_loop(0, D, unroll=8)
        def dbody(d):
            col = (jnp.full((16,), d, jnp.int32) + lane) & 63
            i0 = col >> 3
            i1base = (col & 7) << 7
            for grp in range(8):
                v = plsc.load_gather(gb, [rows[grp], col])
                plsc.store_scatter(sb, [i0, i1base + rows[grp]], v)

    def group(gidx, first):
        bb = gidx // (B_COLS // NTG)
        tg = gidx - bb * (B_COLS // NTG)
        t0 = tg * NTG
        b0 = w_b0 + bb * 128
        bt = w_bt0 + bb
        pltpu.sync_copy(
            xt_hbm.at[pl.ds(t0, NTG), pl.ds(b0, 128)], xblk)
        gather(0, g0, sem_g0).start()
        for tt in range(NTG):
            gb, sg = (g0, sem_g0) if tt % 2 == 0 else (g1, sem_g1)
            sb, ss = (s0, sem_s0) if tt % 2 == 0 else (s1, sem_s1)
            nb, sn = (g1, sem_g1) if tt % 2 == 0 else (g0, sem_g0)
            if tt + 1 < NTG:
                gather(tt + 1, nb, sn).start()
            gather(tt, gb, sg).wait()
            if not (first and tt < 2):
                store(0, 0, sb, ss).wait()
            transpose(gb, sb)
            store(t0 + tt, bt, sb, ss).start()

    group(0, True)

    def body(gidx, carry):
        group(gidx, False)
        return carry

    lax.fori_loop(1, NG, body, 0)

    store(0, 0, s0, sem_s0).wait()
    store(0, 0, s1, sem_s1).wait()


def kernel(x, weight):
    xt = jnp.transpose(x).astype(jnp.int32)
    a = _emb_lookup(xt, weight)
    a5 = a.reshape(B_COLS, 8, 128, 8, 128)
    return jnp.transpose(a5, (2, 4, 0, 1, 3)).reshape(B_ROWS, B_COLS, D)
